# baseline (speedup 1.0000x reference)
import functools

import jax
from jax import lax
from jax.experimental import pallas as pl
from jax.experimental.pallas import tpu as pltpu
from jax.experimental.pallas import tpu_sc as plsc

_NC = 2


@functools.partial(jax.jit, static_argnums=(0, 1, 2))
def _broadcast_rows(B, T, H, W):
    rows_c = T // _NC
    ch = 512 if rows_c % 512 == 0 else rows_c
    n_chunks = rows_c // ch
    mesh = plsc.ScalarSubcoreMesh(axis_name="c", num_cores=_NC)

    @functools.partial(
        pl.kernel,
        mesh=mesh,
        out_type=jax.ShapeDtypeStruct((B, T, H), W.dtype),
        scratch_types=[
            pltpu.VMEM_SHARED((ch, H), W.dtype),
            pltpu.VMEM_SHARED((ch, H), W.dtype),
            pltpu.SemaphoreType.DMA,
            pltpu.SemaphoreType.DMA,
        ],
    )
    def body(w_hbm, out_hbm, buf0, buf1, lsem, ssem):
        c = lax.axis_index("c")
        base = c * rows_c
        bufs = (buf0, buf1)
        loads = [None] * n_chunks
        pending = [[], []]
        loads[0] = pltpu.async_copy(w_hbm.at[pl.ds(base, ch)], bufs[0], lsem)
        for i in range(n_chunks):
            nxt = (i + 1) % 2
            if i + 1 < n_chunks:
                for st in pending[nxt]:
                    st.wait()
                pending[nxt] = []
                loads[i + 1] = pltpu.async_copy(
                    w_hbm.at[pl.ds(base + (i + 1) * ch, ch)], bufs[nxt], lsem)
            loads[i].wait()
            for b in range(B):
                pending[i % 2].append(pltpu.async_copy(
                    bufs[i % 2],
                    out_hbm.at[b].at[pl.ds(base + i * ch, ch)],
                    ssem))
        for lst in pending:
            for st in lst:
                st.wait()

    return body(W)


def kernel(X, W, dim):
    B, T = X.shape
    _, H = W.shape
    return _broadcast_rows(B, T, H, W)


# speedup vs baseline: 1.3921x; 1.3921x over previous
import functools

import jax
import jax.numpy as jnp
from jax import lax
from jax.experimental import pallas as pl
from jax.experimental.pallas import tpu as pltpu
from jax.experimental.pallas import tpu_sc as plsc

_NC = 2
_NS = 16


def _chunk_schedule(rows, ch):
    out = []
    left = rows
    while left > 0:
        c = min(ch, left)
        out.append(c)
        left -= c
    return out


def _pipelined_copy(w_hbm, out_hbm, bufs, lsem, ssem, base, chunks, B):
    starts = [sum(chunks[:i]) for i in range(len(chunks))]
    n = len(chunks)
    loads = [None] * n
    pending = [[], []]
    loads[0] = pltpu.async_copy(
        w_hbm.at[pl.ds(base, chunks[0])],
        bufs[0].at[pl.ds(0, chunks[0])], lsem)
    for i in range(n):
        nxt = (i + 1) % 2
        if i + 1 < n:
            for st in pending[nxt]:
                st.wait()
            pending[nxt] = []
            loads[i + 1] = pltpu.async_copy(
                w_hbm.at[pl.ds(base + starts[i + 1], chunks[i + 1])],
                bufs[nxt].at[pl.ds(0, chunks[i + 1])], lsem)
        loads[i].wait()
        for b in range(B):
            pending[i % 2].append(pltpu.async_copy(
                bufs[i % 2].at[pl.ds(0, chunks[i])],
                out_hbm.at[b].at[pl.ds(base + starts[i], chunks[i])],
                ssem))
    for lst in pending:
        for st in lst:
            st.wait()


@functools.partial(jax.jit, static_argnums=(0, 1, 2))
def _broadcast_rows(B, T, H, W):
    nw = _NC * _NS
    rows_scs = (3 * T // 8) // _NC if T >= nw * 8 else 0
    rows_tec_total = T - _NC * rows_scs
    rows_w = rows_tec_total // nw
    rem = rows_tec_total - rows_w * nw
    tec_base0 = _NC * rows_scs + rem

    v_ch = 48
    s_ch = 256

    smesh = plsc.ScalarSubcoreMesh(axis_name="c", num_cores=_NC)
    vmesh = plsc.VectorSubcoreMesh(
        core_axis_name="c", subcore_axis_name="s",
        num_cores=_NC, num_subcores=_NS,
    )

    def scs_body(w_hbm, out_hbm, sbuf0, sbuf1, slsem, sssem,
                 vbuf0, vbuf1, vlsem, vssem):
        del vbuf0, vbuf1, vlsem, vssem
        c = lax.axis_index("c")
        chunks = _chunk_schedule(rows_scs + rem, s_ch)
        base = jnp.where(c == 0, 0, tec_base0 - (rows_scs + rem))
        _pipelined_copy(w_hbm, out_hbm, (sbuf0, sbuf1), slsem, sssem,
                        base, chunks, B)

    def tec_body(w_hbm, out_hbm, sbuf0, sbuf1, slsem, sssem,
                 vbuf0, vbuf1, vlsem, vssem):
        del sbuf0, sbuf1, slsem, sssem
        wid = lax.axis_index("s") * _NC + lax.axis_index("c")
        base = tec_base0 + wid * rows_w
        chunks = _chunk_schedule(rows_w, v_ch)
        _pipelined_copy(w_hbm, out_hbm, (vbuf0, vbuf1), vlsem, vssem,
                        base, chunks, B)

    run = pl.kernel(
        [scs_body, tec_body],
        out_type=jax.ShapeDtypeStruct((B, T, H), W.dtype),
        mesh=[smesh, vmesh],
        scratch_types=[
            pltpu.VMEM_SHARED((s_ch, H), W.dtype),
            pltpu.VMEM_SHARED((s_ch, H), W.dtype),
            pltpu.SemaphoreType.DMA @ smesh,
            pltpu.SemaphoreType.DMA @ smesh,
            (pltpu.VMEM @ vmesh)((v_ch, H), W.dtype),
            (pltpu.VMEM @ vmesh)((v_ch, H), W.dtype),
            pltpu.SemaphoreType.DMA @ vmesh,
            pltpu.SemaphoreType.DMA @ vmesh,
        ],
    )
    return run(W)


def kernel(X, W, dim):
    B, T = X.shape
    _, H = W.shape
    return _broadcast_rows(B, T, H, W)
